# baseline (speedup 1.0000x reference)
import functools

import jax
import jax.numpy as jnp
from jax import lax
from jax.experimental import pallas as pl
from jax.experimental.pallas import tpu as pltpu
from jax.experimental.pallas import tpu_sc as plsc

S, D, H, E, F = 2048, 768, 12, 8, 3072
DH = D // H
LN_EPS = 1e-5
NEG = -1e30

SB_QKV = 512
QB = 512
SB_RT = 512
BLK = 512
NB = 16
XS = NB * BLK
DW = D // 2
NW = 32
TPW = S // NW
SUB = 32


def _layernorm(x, g, b):
    m = jnp.mean(x, -1, keepdims=True)
    v = jnp.mean(jnp.square(x - m), -1, keepdims=True)
    return (x - m) * jax.lax.rsqrt(v + LN_EPS) * g + b


def _qkv_body(x_ref, g_ref, b_ref, w_ref, bias_ref, o_ref):
    h = _layernorm(x_ref[...], g_ref[...], b_ref[...])
    o = jnp.dot(h.astype(jnp.bfloat16), w_ref[...],
                preferred_element_type=jnp.float32)
    o_ref[...] = (o + bias_ref[...]).astype(jnp.bfloat16)


def _attn_body(q_ref, k_ref, v_ref, o_ref):
    for hh in range(2):
        sl = slice(hh * DH, (hh + 1) * DH)
        s = jax.lax.dot_general(q_ref[:, sl], k_ref[:, sl],
                                (((1,), (1,)), ((), ())),
                                preferred_element_type=jnp.float32)
        s = s * 0.125
        s = s - jnp.max(s, -1, keepdims=True)
        p = jnp.exp(s)
        r = (1.0 / jnp.sum(p, -1, keepdims=True)).astype(jnp.bfloat16)
        o = jnp.dot(p.astype(jnp.bfloat16) * r, v_ref[:, sl],
                    preferred_element_type=jnp.float32)
        o_ref[:, sl] = o.astype(jnp.bfloat16)


def _router_body(x_ref, a_ref, wo_ref, bo_ref, g_ref, b_ref, wg_ref, bg_ref,
                 x2_ref, h2_ref, meta_ref):
    ao = jnp.dot(a_ref[...], wo_ref[...],
                 preferred_element_type=jnp.float32) + bo_ref[...]
    x2 = x_ref[...] + ao
    x2_ref[...] = x2
    h2 = _layernorm(x2, g_ref[...], b_ref[...])
    h2_ref[...] = h2.astype(jnp.bfloat16)
    logits = jnp.dot(h2, wg_ref[...],
                     preferred_element_type=jnp.float32) + bg_ref[...]
    lane = jax.lax.broadcasted_iota(jnp.int32, logits.shape, 1)
    m1 = jnp.max(logits, -1, keepdims=True)
    i1 = jnp.min(jnp.where(logits == m1, lane, 128), -1, keepdims=True)
    l2 = jnp.where(lane == i1, NEG, logits)
    m2 = jnp.max(l2, -1, keepdims=True)
    i2 = jnp.min(jnp.where(l2 == m2, lane, 128), -1, keepdims=True)
    w2 = 1.0 / (1.0 + jnp.exp(m1 - m2))
    w1 = 1.0 - w2
    meta = jnp.where(lane == 0, i1.astype(jnp.float32), 0.0)
    meta = meta + jnp.where(lane == 1, i2.astype(jnp.float32), 0.0)
    meta = meta + jnp.where(lane == 2, w1, 0.0)
    meta = meta + jnp.where(lane == 3, w2, 0.0)
    meta_ref[...] = meta


def _offsets_body(meta_ref, p0_ref, p1_ref, bex_ref, bval_ref):
    meta = meta_ref[...]
    i1 = meta[:, 0:1].astype(jnp.int32)
    i2 = meta[:, 1:2].astype(jnp.int32)
    lane = jax.lax.broadcasted_iota(jnp.int32, (S, 128), 1)
    m = ((lane == i1) | (lane == i2)).astype(jnp.float32)
    r_iota = jax.lax.broadcasted_iota(jnp.int32, (128, 128), 0)
    c_iota = jax.lax.broadcasted_iota(jnp.int32, (128, 128), 1)
    tri = (r_iota >= c_iota).astype(jnp.float32)
    run = jnp.zeros((1, 128), jnp.float32)
    ranks = []
    for c in range(S // 128):
        blk = m[c * 128:(c + 1) * 128, :]
        intra = jnp.dot(tri, blk, preferred_element_type=jnp.float32)
        ranks.append(run + intra - blk)
        run = run + intra[127:128, :]
    rank = jnp.concatenate(ranks, axis=0)
    padded = (((run.astype(jnp.int32) + BLK - 1) // BLK) * BLK)
    padf = padded.astype(jnp.float32)
    upper = (r_iota < c_iota).astype(jnp.float32)
    poff = jnp.dot(padf, upper, preferred_element_type=jnp.float32)
    pend = (poff + padf).astype(jnp.int32)
    posmat = poff + rank
    pos0 = jnp.sum(jnp.where(lane == i1, posmat, 0.0), -1, keepdims=True)
    pos1 = jnp.sum(jnp.where(lane == i2, posmat, 0.0), -1, keepdims=True)
    p0_ref[...] = pos0.astype(jnp.int32)
    p1_ref[...] = pos1.astype(jnp.int32)
    bi = r_iota * BLK
    a = ((bi >= pend) & (c_iota < E)).astype(jnp.float32)
    be = jnp.sum(a, -1, keepdims=True).astype(jnp.int32)
    totsum = jnp.sum(padf, -1, keepdims=True).astype(jnp.int32)
    bex_ref[...] = jnp.minimum(be, E - 1)
    bval_ref[...] = (bi[:, 0:1] < totsum).astype(jnp.int32)


def _ffn_body(bex_ref, bval_ref, xs_ref, ws_ref, w1_ref, b1_ref, w2_ref,
              b2_ref, ys_ref):
    i = pl.program_id(0)

    @pl.when(bval_ref[i] == 1)
    def _():
        t = jnp.dot(xs_ref[...], w1_ref[0],
                    preferred_element_type=jnp.float32) + b1_ref[0]
        t = jax.nn.gelu(t.astype(jnp.bfloat16))
        y = jnp.dot(t, w2_ref[0],
                    preferred_element_type=jnp.float32) + b2_ref[0]
        ys_ref[...] = y * ws_ref[:, 0:1]


def _sc_dispatch(h2i_hbm, w0_hbm, w1_hbm, p0_hbm, p1_hbm, xs_hbm, ws_hbm,
                 p0_v, p1_v, rows_v, w0_v, w1_v, sem):
    wid = lax.axis_index("s") * 2 + lax.axis_index("c")
    base = wid * TPW
    pltpu.sync_copy(p0_hbm.at[pl.ds(base, TPW)], p0_v)
    pltpu.sync_copy(p1_hbm.at[pl.ds(base, TPW)], p1_v)
    pltpu.sync_copy(h2i_hbm.at[pl.ds(base, TPW)], rows_v)
    pltpu.sync_copy(w0_hbm.at[pl.ds(base, TPW)], w0_v)
    pltpu.sync_copy(w1_hbm.at[pl.ds(base, TPW)], w1_v)
    pltpu.async_copy(rows_v, xs_hbm.at[p0_v], sem).wait()
    pltpu.async_copy(rows_v, xs_hbm.at[p1_v], sem).wait()
    pltpu.async_copy(w0_v, ws_hbm.at[p0_v], sem).wait()
    pltpu.async_copy(w1_v, ws_hbm.at[p1_v], sem).wait()


def _sc_combine(ys_hbm, x2_hbm, p0_hbm, p1_hbm, out_hbm,
                p0_v, p1_v, g0_v, g1_v, acc_v, sem):
    wid = lax.axis_index("s") * 2 + lax.axis_index("c")
    for it in range(TPW // SUB):
        base = wid * TPW + it * SUB
        pltpu.sync_copy(p0_hbm.at[pl.ds(base, SUB)], p0_v)
        pltpu.sync_copy(p1_hbm.at[pl.ds(base, SUB)], p1_v)
        pltpu.sync_copy(x2_hbm.at[pl.ds(base, SUB)], acc_v)
        pltpu.async_copy(ys_hbm.at[p0_v], g0_v, sem).wait()
        pltpu.async_copy(ys_hbm.at[p1_v], g1_v, sem).wait()
        for r in range(SUB):
            def body(j, _, _r=r):
                sl = pl.ds(j * 16, 16)
                acc_v[_r, sl] = (acc_v[_r, sl] + g0_v[_r, sl]
                                 + g1_v[_r, sl])
                return 0
            lax.fori_loop(0, D // 16, body, 0)
        pltpu.sync_copy(acc_v, out_hbm.at[pl.ds(base, SUB)])


def kernel(x, gamma1, beta1, Wq, bq, Wk, bk, Wv, bv, Wo, bo,
           gamma2, beta2, Wg, bg, W1, b1, W2, b2):
    xs_in = x.reshape(S, D)
    wqkv = jnp.concatenate([Wq, Wk, Wv], axis=1).astype(jnp.bfloat16)
    bqkv = jnp.concatenate([bq, bk, bv]).reshape(1, 3 * D)

    qkv = pl.pallas_call(
        _qkv_body,
        grid=(S // SB_QKV,),
        in_specs=[
            pl.BlockSpec((SB_QKV, D), lambda i: (i, 0)),
            pl.BlockSpec((1, D), lambda i: (0, 0)),
            pl.BlockSpec((1, D), lambda i: (0, 0)),
            pl.BlockSpec((D, 3 * D), lambda i: (0, 0)),
            pl.BlockSpec((1, 3 * D), lambda i: (0, 0)),
        ],
        out_specs=pl.BlockSpec((SB_QKV, 3 * D), lambda i: (i, 0)),
        out_shape=jax.ShapeDtypeStruct((S, 3 * D), jnp.bfloat16),
    )(xs_in, gamma1.reshape(1, D), beta1.reshape(1, D), wqkv, bqkv)

    nhb = D // 128
    attn = pl.pallas_call(
        _attn_body,
        grid=(H // 2, S // QB),
        in_specs=[
            pl.BlockSpec((QB, 128), lambda g, i: (i, g)),
            pl.BlockSpec((S, 128), lambda g, i: (0, nhb + g)),
            pl.BlockSpec((S, 128), lambda g, i: (0, 2 * nhb + g)),
        ],
        out_specs=pl.BlockSpec((QB, 128), lambda g, i: (i, g)),
        out_shape=jax.ShapeDtypeStruct((S, D), jnp.bfloat16),
    )(qkv, qkv, qkv)

    wgp = jnp.zeros((D, 128), jnp.float32).at[:, :E].set(Wg)
    bgp = jnp.full((1, 128), NEG, jnp.float32).at[0, :E].set(bg)
    x2, h2, meta = pl.pallas_call(
        _router_body,
        grid=(S // SB_RT,),
        in_specs=[
            pl.BlockSpec((SB_RT, D), lambda i: (i, 0)),
            pl.BlockSpec((SB_RT, D), lambda i: (i, 0)),
            pl.BlockSpec((D, D), lambda i: (0, 0)),
            pl.BlockSpec((1, D), lambda i: (0, 0)),
            pl.BlockSpec((1, D), lambda i: (0, 0)),
            pl.BlockSpec((1, D), lambda i: (0, 0)),
            pl.BlockSpec((D, 128), lambda i: (0, 0)),
            pl.BlockSpec((1, 128), lambda i: (0, 0)),
        ],
        out_specs=[
            pl.BlockSpec((SB_RT, D), lambda i: (i, 0)),
            pl.BlockSpec((SB_RT, D), lambda i: (i, 0)),
            pl.BlockSpec((SB_RT, 128), lambda i: (i, 0)),
        ],
        out_shape=[
            jax.ShapeDtypeStruct((S, D), jnp.float32),
            jax.ShapeDtypeStruct((S, D), jnp.bfloat16),
            jax.ShapeDtypeStruct((S, 128), jnp.float32),
        ],
    )(xs_in, attn, Wo.astype(jnp.bfloat16), bo.reshape(1, D),
      gamma2.reshape(1, D), beta2.reshape(1, D), wgp, bgp)

    p0m, p1m, bexm, bvalm = pl.pallas_call(
        _offsets_body,
        grid=(1,),
        in_specs=[pl.BlockSpec((S, 128), lambda i: (0, 0))],
        out_specs=[
            pl.BlockSpec((S, 1), lambda i: (0, 0)),
            pl.BlockSpec((S, 1), lambda i: (0, 0)),
            pl.BlockSpec((128, 1), lambda i: (0, 0)),
            pl.BlockSpec((128, 1), lambda i: (0, 0)),
        ],
        out_shape=[
            jax.ShapeDtypeStruct((S, 1), jnp.int32),
            jax.ShapeDtypeStruct((S, 1), jnp.int32),
            jax.ShapeDtypeStruct((128, 1), jnp.int32),
            jax.ShapeDtypeStruct((128, 1), jnp.int32),
        ],
    )(meta)
    pos0 = p0m.reshape(S)
    pos1 = p1m.reshape(S)
    bex = bexm.reshape(128)[:NB]
    bval = bvalm.reshape(128)[:NB]

    h2i = jax.lax.bitcast_convert_type(h2.reshape(S, DW, 2), jnp.int32)
    w0r = jnp.broadcast_to(meta[:, 2:3], (S, 128))
    w1r = jnp.broadcast_to(meta[:, 3:4], (S, 128))

    mesh = plsc.VectorSubcoreMesh(core_axis_name="c", subcore_axis_name="s",
                                  num_cores=2, num_subcores=16)
    xs_i, ws = pl.kernel(
        _sc_dispatch,
        out_type=[jax.ShapeDtypeStruct((XS, DW), jnp.int32),
                  jax.ShapeDtypeStruct((XS, 128), jnp.float32)],
        mesh=mesh,
        scratch_types=[
            pltpu.VMEM((TPW,), jnp.int32),
            pltpu.VMEM((TPW,), jnp.int32),
            pltpu.VMEM((TPW, DW), jnp.int32),
            pltpu.VMEM((TPW, 128), jnp.float32),
            pltpu.VMEM((TPW, 128), jnp.float32),
            pltpu.SemaphoreType.DMA,
        ],
    )(h2i, w0r, w1r, pos0, pos1)
    xs = jax.lax.bitcast_convert_type(xs_i, jnp.bfloat16).reshape(XS, D)

    ys = pl.pallas_call(
        _ffn_body,
        grid_spec=pltpu.PrefetchScalarGridSpec(
            num_scalar_prefetch=2,
            grid=(NB,),
            in_specs=[
                pl.BlockSpec((BLK, D), lambda i, bex, bval: (i, 0)),
                pl.BlockSpec((BLK, 128), lambda i, bex, bval: (i, 0)),
                pl.BlockSpec((1, D, F), lambda i, bex, bval: (bex[i], 0, 0)),
                pl.BlockSpec((1, 1, F), lambda i, bex, bval: (bex[i], 0, 0)),
                pl.BlockSpec((1, F, D), lambda i, bex, bval: (bex[i], 0, 0)),
                pl.BlockSpec((1, 1, D), lambda i, bex, bval: (bex[i], 0, 0)),
            ],
            out_specs=pl.BlockSpec((BLK, D), lambda i, bex, bval: (i, 0)),
        ),
        out_shape=jax.ShapeDtypeStruct((XS, D), jnp.float32),
    )(bex, bval, xs, ws, W1.astype(jnp.bfloat16), b1.reshape(E, 1, F),
      W2.astype(jnp.bfloat16), b2.reshape(E, 1, D))

    out = pl.kernel(
        _sc_combine,
        out_type=jax.ShapeDtypeStruct((S, D), jnp.float32),
        mesh=mesh,
        scratch_types=[
            pltpu.VMEM((SUB,), jnp.int32),
            pltpu.VMEM((SUB,), jnp.int32),
            pltpu.VMEM((SUB, D), jnp.float32),
            pltpu.VMEM((SUB, D), jnp.float32),
            pltpu.VMEM((SUB, D), jnp.float32),
            pltpu.SemaphoreType.DMA,
        ],
    )(ys, x2, pos0, pos1)

    return out.reshape(1, S, D)
